# baseline (speedup 1.0000x reference)
import functools

import jax
import jax.numpy as jnp
from jax import lax
from jax.experimental import pallas as pl
from jax.experimental.pallas import tpu as pltpu
from jax.experimental.pallas import tpu_sc as plsc

VOCAB = 1000000
DIM = 64
BATCH = 4096
SEQ = 50
ENT = 8
OUTSEQ = ENT + SEQ

NC = 2
NS = 16
NW = NC * NS
BPW = BATCH // NW
NB = 8
NCHUNK = BPW // NB


def _sc_kernel(idx_hbm, clip_hbm, table_hbm, out_hbm, idx_v, buf_v, sem):
    wid = lax.axis_index("s") * NC + lax.axis_index("c")

    def chunk_body(c, carry):
        b0 = wid * BPW + c * NB
        pltpu.sync_copy(idx_hbm.at[pl.ds(b0, NB)], idx_v)
        pltpu.sync_copy(clip_hbm.at[pl.ds(b0, NB)],
                        buf_v.at[:, pl.ds(0, ENT)])
        copies = []
        for i in range(NB):
            copies.append(pltpu.async_copy(
                table_hbm.at[idx_v.at[i]],
                buf_v.at[i, pl.ds(ENT, SEQ)],
                sem))
        for cp in copies:
            cp.wait()
        pltpu.sync_copy(buf_v, out_hbm.at[pl.ds(b0, NB)])
        return carry

    lax.fori_loop(0, NCHUNK, chunk_body, 0)


@jax.jit
def _run(idx2d, clip_entity, table):
    mesh = plsc.VectorSubcoreMesh(core_axis_name="c", subcore_axis_name="s")
    kern = functools.partial(
        pl.kernel,
        mesh=mesh,
        compiler_params=pltpu.CompilerParams(use_tc_tiling_on_sc=False),
        out_type=jax.ShapeDtypeStruct((BATCH, OUTSEQ, DIM), jnp.float32),
        scratch_types=[
            pltpu.VMEM((NB, SEQ), jnp.int32),
            pltpu.VMEM((NB, OUTSEQ, DIM), jnp.float32),
            pltpu.SemaphoreType.DMA,
        ],
    )(_sc_kernel)
    return kern(idx2d, clip_entity, table)


def kernel(g_tokens_ids, clip_entity, table):
    return _run(g_tokens_ids.astype(jnp.int32), clip_entity, table)


# speedup vs baseline: 1.0296x; 1.0296x over previous
import functools

import jax
import jax.numpy as jnp
from jax import lax
from jax.experimental import pallas as pl
from jax.experimental.pallas import tpu as pltpu
from jax.experimental.pallas import tpu_sc as plsc

VOCAB = 1000000
DIM = 64
BATCH = 4096
SEQ = 50
ENT = 8
OUTSEQ = ENT + SEQ

NC = 2
NS = 16
NW = NC * NS
BPW = BATCH // NW
NB = 4
NCHUNK = BPW // NB
K = 4
NITER = NCHUNK // K


def _sc_kernel(idx_hbm, clip_hbm, table_hbm, out_hbm, idx_v, buf_v,
               sem_g, sem_w):
    wid = lax.axis_index("s") * NC + lax.axis_index("c")
    base = wid * BPW

    pltpu.sync_copy(idx_hbm.at[pl.ds(base, BPW)], idx_v)

    def fire_chunk(c, k):
        b0 = base + c * NB
        pltpu.async_copy(clip_hbm.at[pl.ds(b0, NB)],
                         buf_v.at[k, :, pl.ds(0, ENT)], sem_g.at[k])
        for i in range(NB):
            pltpu.async_copy(table_hbm.at[idx_v.at[c * NB + i]],
                             buf_v.at[k, i, pl.ds(ENT, SEQ)], sem_g.at[k])

    def wait_chunk(k):
        pltpu.make_async_copy(clip_hbm.at[pl.ds(base, NB)],
                              buf_v.at[k, :, pl.ds(0, ENT)],
                              sem_g.at[k]).wait()
        for i in range(NB):
            pltpu.make_async_copy(table_hbm.at[idx_v.at[i]],
                                  buf_v.at[k, i, pl.ds(ENT, SEQ)],
                                  sem_g.at[k]).wait()

    def fire_write(c, k):
        pltpu.async_copy(buf_v.at[k], out_hbm.at[pl.ds(base + c * NB, NB)],
                         sem_w.at[k])

    def wait_write(k):
        pltpu.make_async_copy(buf_v.at[k], out_hbm.at[pl.ds(base, NB)],
                              sem_w.at[k]).wait()

    def body(g, carry):
        for k in range(K):
            c = g * K + k

            @pl.when(g > 0)
            def _():
                wait_write(k)

            fire_chunk(c, k)
        for k in range(K):
            wait_chunk(k)
            fire_write(g * K + k, k)
        return carry

    lax.fori_loop(0, NITER, body, 0)
    for k in range(K):
        wait_write(k)


@jax.jit
def _run(idx2d, clip_entity, table):
    mesh = plsc.VectorSubcoreMesh(core_axis_name="c", subcore_axis_name="s")
    kern = functools.partial(
        pl.kernel,
        mesh=mesh,
        compiler_params=pltpu.CompilerParams(use_tc_tiling_on_sc=False),
        out_type=jax.ShapeDtypeStruct((BATCH, OUTSEQ, DIM), jnp.float32),
        scratch_types=[
            pltpu.VMEM((BPW, SEQ), jnp.int32),
            pltpu.VMEM((K, NB, OUTSEQ, DIM), jnp.float32),
            pltpu.SemaphoreType.DMA((K,)),
            pltpu.SemaphoreType.DMA((K,)),
        ],
    )(_sc_kernel)
    return kern(idx2d, clip_entity, table)


def kernel(g_tokens_ids, clip_entity, table):
    return _run(g_tokens_ids.astype(jnp.int32), clip_entity, table)
